# speedup vs baseline: 2.6784x; 2.6784x over previous
import functools

import jax
import jax.numpy as jnp
from jax.experimental import pallas as pl
from jax.experimental.pallas import tpu as pltpu

SCALE_ = 30.0
MARGIN_ = 0.2
B_ = 1024
C_ = 100000
BC_ = 2048
SHIFT_ = 30.0


def _loss_body(label_ref, cos_ref, loss_ref, sacc_ref, tacc_ref, *, nblk):
    j = pl.program_id(0)

    @pl.when(j == 0)
    def _init():
        sacc_ref[...] = jnp.zeros_like(sacc_ref)
        tacc_ref[...] = jnp.zeros_like(tacc_ref)

    x = cos_ref[...]
    col = j * BC_ + jax.lax.broadcasted_iota(jnp.int32, (B_, BC_), 1)
    lbl = label_ref[...]
    e = jnp.exp(x * SCALE_ - SHIFT_)
    e = jnp.where(col < C_, e, 0.0)
    sacc_ref[...] += e
    tacc_ref[...] += jnp.where(col == lbl, x, 0.0)

    @pl.when(j == nblk - 1)
    def _finish():
        s = jnp.sum(sacc_ref[...], axis=1, keepdims=True)
        t = jnp.sum(tacc_ref[...], axis=1, keepdims=True)
        tm = (t - MARGIN_) * SCALE_
        s = s - jnp.exp(t * SCALE_ - SHIFT_) + jnp.exp(tm - SHIFT_)
        nll = SHIFT_ + jnp.log(s) - tm
        loss_ref[0, 0] = jnp.sum(nll) / B_


def kernel(cosine, label):
    nblk = pl.cdiv(C_, BC_)
    loss = pl.pallas_call(
        functools.partial(_loss_body, nblk=nblk),
        grid=(nblk,),
        in_specs=[
            pl.BlockSpec((B_, 1), lambda j: (0, 0)),
            pl.BlockSpec((B_, BC_), lambda j: (0, j)),
        ],
        out_specs=pl.BlockSpec((1, 1), lambda j: (0, 0), memory_space=pltpu.SMEM),
        out_shape=jax.ShapeDtypeStruct((1, 1), jnp.float32),
        scratch_shapes=[
            pltpu.VMEM((B_, BC_), jnp.float32),
            pltpu.VMEM((B_, BC_), jnp.float32),
        ],
    )(label.reshape(B_, 1), cosine)
    return loss[0, 0]


# speedup vs baseline: 2.9518x; 1.1021x over previous
import functools

import jax
import jax.numpy as jnp
from jax.experimental import pallas as pl
from jax.experimental.pallas import tpu as pltpu

SCALE_ = 30.0
MARGIN_ = 0.2
B_ = 1024
C_ = 100000
BC_ = 2048
NFOLD_ = BC_ // 128
LOG2E_ = 1.4426950408889634
K_ = SCALE_ * LOG2E_


def _fold128(v):
    acc = v[:, 0:128]
    for k in range(1, v.shape[1] // 128):
        acc = acc + v[:, k * 128:(k + 1) * 128]
    return acc


def _loss_body(label_ref, cos_ref, loss_ref, sacc_ref, tacc_ref, *, nblk):
    j = pl.program_id(0)

    @pl.when(j == 0)
    def _init():
        sacc_ref[...] = jnp.zeros_like(sacc_ref)
        tacc_ref[...] = jnp.zeros_like(tacc_ref)

    x = cos_ref[...]
    lbl_loc = label_ref[...] - j * BC_
    is_t = jax.lax.broadcasted_iota(jnp.int32, (B_, BC_), 1) == lbl_loc
    e = jnp.exp2(x * K_ - K_)

    @pl.when(j < nblk - 1)
    def _main():
        sacc_ref[...] += _fold128(e)
        tacc_ref[...] += _fold128(jnp.where(is_t, x, 0.0))

    @pl.when(j == nblk - 1)
    def _last():
        valid = jax.lax.broadcasted_iota(jnp.int32, (B_, BC_), 1) < (C_ - (nblk - 1) * BC_)
        sacc_ref[...] += _fold128(jnp.where(valid, e, 0.0))
        tacc_ref[...] += _fold128(jnp.where(is_t, x, 0.0))
        s = jnp.sum(sacc_ref[...], axis=1, keepdims=True)
        t = jnp.sum(tacc_ref[...], axis=1, keepdims=True)
        tm = (t - MARGIN_) * SCALE_
        s = s - jnp.exp2(t * K_ - K_) + jnp.exp2(tm * LOG2E_ - K_)
        nll = SCALE_ + jnp.log(s) - tm
        loss_ref[0, 0] = jnp.sum(nll) / B_


def kernel(cosine, label):
    nblk = pl.cdiv(C_, BC_)
    loss = pl.pallas_call(
        functools.partial(_loss_body, nblk=nblk),
        grid=(nblk,),
        in_specs=[
            pl.BlockSpec((B_, 1), lambda j: (0, 0)),
            pl.BlockSpec((B_, BC_), lambda j: (0, j)),
        ],
        out_specs=pl.BlockSpec((1, 1), lambda j: (0, 0), memory_space=pltpu.SMEM),
        out_shape=jax.ShapeDtypeStruct((1, 1), jnp.float32),
        scratch_shapes=[
            pltpu.VMEM((B_, 128), jnp.float32),
            pltpu.VMEM((B_, 128), jnp.float32),
        ],
    )(label.reshape(B_, 1), cosine)
    return loss[0, 0]


# speedup vs baseline: 2.9647x; 1.0044x over previous
import functools

import jax
import jax.numpy as jnp
from jax import lax
from jax.experimental import pallas as pl
from jax.experimental.pallas import tpu as pltpu
from jax.experimental.pallas import tpu_sc as plsc

SCALE_ = 30.0
MARGIN_ = 0.2
B_ = 1024
C_ = 100000
LOG2E_ = 1.4426950408889634
K_ = SCALE_ * LOG2E_

RTC_ = 512
RSC_ = B_ - RTC_
BC_ = 2048

NW_ = 32
RPW_ = RSC_ // NW_
NRG_ = RPW_ // 8
CHC_ = 6144
NFULL_ = 16
CSC_ = NFULL_ * CHC_
TAIL_ = C_ - CSC_
NTBLK_ = pl.cdiv(C_, BC_) - 1


def _sc_body(cos_hbm, lbl_hbm, s_out, t_out, buf0, buf1, lbl_v, srow_v, trow_v,
             sem0, sem1):
    wid = lax.axis_index("s") * 2 + lax.axis_index("c")
    r0 = RTC_ + wid * RPW_
    pltpu.sync_copy(lbl_hbm.at[pl.ds(wid * RPW_, RPW_)], lbl_v)

    bufs = (buf0, buf1)
    sems = (sem0, sem1)
    ntask = NRG_ * NFULL_

    def start(t):
        rg, c = t // NFULL_, t % NFULL_
        return pltpu.async_copy(
            cos_hbm.at[pl.ds(r0 + rg * 8, 8), pl.ds(c * CHC_, CHC_)],
            bufs[t % 2], sems[t % 2])

    pending = start(0)
    accs = taccs = None
    for t in range(ntask):
        rg, c = t // NFULL_, t % NFULL_
        if t + 1 < ntask:
            nxt = start(t + 1)
        pending.wait()
        pending = nxt if t + 1 < ntask else None
        buf = bufs[t % 2]
        if c == 0:
            accs = [jnp.zeros((16,), jnp.float32) for _ in range(8)]
            taccs = [jnp.zeros((16,), jnp.float32) for _ in range(8)]

        def step(i, a):
            out = []
            for rr in range(8):
                x = buf[rr, pl.ds(i * 16, 16)]
                out.append(a[rr] + jnp.exp(x * SCALE_ - SCALE_))
            return tuple(out)

        accs = list(lax.fori_loop(0, CHC_ // 16, step, tuple(accs)))

        c0 = c * CHC_
        for rr in range(8):
            lblv = plsc.load_gather(lbl_v, [jnp.full((16,), rg * 8 + rr, jnp.int32)])
            loc = lblv - c0
            in_chunk = (loc >= 0) & (loc < CHC_)
            g = plsc.load_gather(buf, [jnp.full((16,), rr, jnp.int32),
                                       jnp.clip(loc, 0, CHC_ - 1)])
            taccs[rr] = taccs[rr] + jnp.where(in_chunk, g, 0.0)

        if c == NFULL_ - 1:
            for rr in range(8):
                srow_v[pl.ds((rg * 8 + rr) * 16, 16)] = accs[rr]
                trow_v[pl.ds((rg * 8 + rr) * 16, 16)] = taccs[rr]
    pltpu.sync_copy(srow_v, s_out.at[pl.ds(wid * RPW_ * 16, RPW_ * 16)])
    pltpu.sync_copy(trow_v, t_out.at[pl.ds(wid * RPW_ * 16, RPW_ * 16)])


def _make_sc():
    mesh = plsc.VectorSubcoreMesh(core_axis_name="c", subcore_axis_name="s")
    return pl.kernel(
        _sc_body,
        out_type=(
            jax.ShapeDtypeStruct((RSC_ * 16,), jnp.float32),
            jax.ShapeDtypeStruct((RSC_ * 16,), jnp.float32),
        ),
        mesh=mesh,
        compiler_params=pltpu.CompilerParams(needs_layout_passes=False),
        scratch_types=[
            pltpu.VMEM((8, CHC_), jnp.float32),
            pltpu.VMEM((8, CHC_), jnp.float32),
            pltpu.VMEM((RPW_,), jnp.int32),
            pltpu.VMEM((RPW_ * 16,), jnp.float32),
            pltpu.VMEM((RPW_ * 16,), jnp.float32),
            pltpu.SemaphoreType.DMA,
            pltpu.SemaphoreType.DMA,
        ],
    )


def _fold128(v):
    acc = v[:, 0:128]
    for k in range(1, v.shape[1] // 128):
        acc = acc + v[:, k * 128:(k + 1) * 128]
    return acc


def _tc_body(label_ref, cos_ref, out_ref, sacc_ref, tacc_ref, *, nblk):
    j = pl.program_id(0)

    @pl.when(j == 0)
    def _init():
        sacc_ref[...] = jnp.zeros_like(sacc_ref)
        tacc_ref[...] = jnp.zeros_like(tacc_ref)

    x = cos_ref[...]
    lbl_loc = label_ref[...] - j * BC_
    is_t = jax.lax.broadcasted_iota(jnp.int32, (RTC_, BC_), 1) == lbl_loc
    e = jnp.exp2(x * K_ - K_)

    @pl.when(j < nblk - 1)
    def _main():
        sacc_ref[...] += _fold128(e)
        tacc_ref[...] += _fold128(jnp.where(is_t, x, 0.0))

    @pl.when(j == nblk - 1)
    def _last():
        valid = jax.lax.broadcasted_iota(jnp.int32, (RTC_, BC_), 1) < (C_ - (nblk - 1) * BC_)
        sacc_ref[...] += _fold128(jnp.where(valid, e, 0.0))
        tacc_ref[...] += _fold128(jnp.where(is_t, x, 0.0))
        s = jnp.sum(sacc_ref[...], axis=1, keepdims=True)
        t = jnp.sum(tacc_ref[...], axis=1, keepdims=True)
        tm = (t - MARGIN_) * SCALE_
        s = s - jnp.exp2(t * K_ - K_) + jnp.exp2(tm * LOG2E_ - K_)
        nll = SCALE_ + jnp.log(s) - tm
        out_ref[0, 0] = jnp.sum(nll)


def _tc_main(cosine, label):
    nblk = pl.cdiv(C_, BC_)
    return pl.pallas_call(
        functools.partial(_tc_body, nblk=nblk),
        grid=(nblk,),
        in_specs=[
            pl.BlockSpec((RTC_, 1), lambda j: (0, 0)),
            pl.BlockSpec((RTC_, BC_), lambda j: (0, j)),
        ],
        out_specs=pl.BlockSpec((1, 1), lambda j: (0, 0), memory_space=pltpu.SMEM),
        out_shape=jax.ShapeDtypeStruct((1, 1), jnp.float32),
        scratch_shapes=[
            pltpu.VMEM((RTC_, 128), jnp.float32),
            pltpu.VMEM((RTC_, 128), jnp.float32),
        ],
    )(label[:RTC_].reshape(RTC_, 1), cosine)


def _combine_body(s_slab_ref, t_slab_ref, lbl_ref, cos_ref, tc_part_ref, loss_ref):
    x = cos_ref[...]
    col = NTBLK_ * BC_ + jax.lax.broadcasted_iota(jnp.int32, (RSC_, BC_), 1)
    lbl = lbl_ref[...]
    e = jnp.where(col < C_, jnp.exp2(x * K_ - K_), 0.0)
    s_tail = jnp.sum(e, axis=1, keepdims=True)
    t_tail = jnp.sum(jnp.where(col == lbl, x, 0.0), axis=1, keepdims=True)
    s = jnp.sum(s_slab_ref[...], axis=1, keepdims=True) + s_tail
    t = t_slab_ref[:, 0:1] + t_tail
    tm = (t - MARGIN_) * SCALE_
    s = s - jnp.exp2(t * K_ - K_) + jnp.exp2(tm * LOG2E_ - K_)
    nll = SCALE_ + jnp.log(s) - tm
    loss_ref[0, 0] = (jnp.sum(nll) + tc_part_ref[0, 0]) / B_


def _combine(s_slab, t_slab, label, cosine, tc_part):
    return pl.pallas_call(
        _combine_body,
        grid=(1,),
        in_specs=[
            pl.BlockSpec((RSC_, 16), lambda j: (0, 0)),
            pl.BlockSpec((RSC_, 16), lambda j: (0, 0)),
            pl.BlockSpec((RSC_, 1), lambda j: (RTC_ // RSC_, 0)),
            pl.BlockSpec((RSC_, BC_), lambda j: (RTC_ // RSC_, NTBLK_)),
            pl.BlockSpec((1, 1), lambda j: (0, 0), memory_space=pltpu.SMEM),
        ],
        out_specs=pl.BlockSpec((1, 1), lambda j: (0, 0), memory_space=pltpu.SMEM),
        out_shape=jax.ShapeDtypeStruct((1, 1), jnp.float32),
    )(s_slab, t_slab, label, cosine, tc_part)


def kernel(cosine, label):
    sc_fn = _make_sc()
    s_slab, t_slab = sc_fn(cosine, label[RTC_:])
    tc_part = _tc_main(cosine, label)
    loss = _combine(
        s_slab.reshape(RSC_, 16), t_slab.reshape(RSC_, 16),
        label.reshape(B_, 1), cosine, tc_part)
    return loss[0, 0]


# speedup vs baseline: 2.9668x; 1.0007x over previous
import functools

import jax
import jax.numpy as jnp
from jax import lax
from jax.experimental import pallas as pl
from jax.experimental.pallas import tpu as pltpu
from jax.experimental.pallas import tpu_sc as plsc

SCALE_ = 30.0
MARGIN_ = 0.2
B_ = 1024
C_ = 100000
LOG2E_ = 1.4426950408889634
K_ = SCALE_ * LOG2E_

RTC_ = 512
RSC_ = B_ - RTC_
BC_ = 2048

NW_ = 32
RPW_ = RSC_ // NW_
NRG_ = RPW_ // 8
CHC_ = 6144
NFULL_ = 16
CSC_ = NFULL_ * CHC_
TAIL_ = C_ - CSC_
NTBLK_ = pl.cdiv(C_, BC_) - 1


def _sc_body(cos_hbm, lbl_hbm, s_out, t_out, buf0, buf1, lbl_v, srow_v, trow_v,
             sem0, sem1):
    wid = lax.axis_index("s") * 2 + lax.axis_index("c")
    r0 = RTC_ + wid * RPW_
    pltpu.sync_copy(lbl_hbm.at[pl.ds(wid * RPW_, RPW_)], lbl_v)

    bufs = (buf0, buf1)
    sems = (sem0, sem1)
    ntask = NRG_ * NFULL_

    def start(t):
        rg, c = t // NFULL_, t % NFULL_
        return pltpu.async_copy(
            cos_hbm.at[pl.ds(r0 + rg * 8, 8), pl.ds(c * CHC_, CHC_)],
            bufs[t % 2], sems[t % 2])

    pending = start(0)
    accs = taccs = None
    for t in range(ntask):
        rg, c = t // NFULL_, t % NFULL_
        if t + 1 < ntask:
            nxt = start(t + 1)
        pending.wait()
        pending = nxt if t + 1 < ntask else None
        buf = bufs[t % 2]
        if c == 0:
            accs = [jnp.zeros((16,), jnp.float32) for _ in range(8)]
            taccs = [jnp.zeros((16,), jnp.float32) for _ in range(8)]

        def step(i, a):
            out = []
            for rr in range(8):
                x = buf[rr, pl.ds(i * 16, 16)]
                out.append(a[rr] + jnp.exp(x * SCALE_ - SCALE_))
            return tuple(out)

        accs = list(lax.fori_loop(0, CHC_ // 16, step, tuple(accs)))

        c0 = c * CHC_
        for rr in range(8):
            lblv = plsc.load_gather(lbl_v, [jnp.full((16,), rg * 8 + rr, jnp.int32)])
            loc = lblv - c0
            in_chunk = (loc >= 0) & (loc < CHC_)
            g = plsc.load_gather(buf, [jnp.full((16,), rr, jnp.int32),
                                       jnp.clip(loc, 0, CHC_ - 1)])
            taccs[rr] = taccs[rr] + jnp.where(in_chunk, g, 0.0)

        if c == NFULL_ - 1:
            for rr in range(8):
                srow_v[pl.ds((rg * 8 + rr) * 16, 16)] = accs[rr]
                trow_v[pl.ds((rg * 8 + rr) * 16, 16)] = taccs[rr]
    pltpu.sync_copy(srow_v, s_out.at[pl.ds(wid * RPW_ * 16, RPW_ * 16)])
    pltpu.sync_copy(trow_v, t_out.at[pl.ds(wid * RPW_ * 16, RPW_ * 16)])


def _make_sc():
    mesh = plsc.VectorSubcoreMesh(core_axis_name="c", subcore_axis_name="s")
    return pl.kernel(
        _sc_body,
        out_type=(
            jax.ShapeDtypeStruct((RSC_ * 16,), jnp.float32),
            jax.ShapeDtypeStruct((RSC_ * 16,), jnp.float32),
        ),
        mesh=mesh,
        compiler_params=pltpu.CompilerParams(needs_layout_passes=False, use_tc_tiling_on_sc=True),
        scratch_types=[
            pltpu.VMEM((8, CHC_), jnp.float32),
            pltpu.VMEM((8, CHC_), jnp.float32),
            pltpu.VMEM((RPW_,), jnp.int32),
            pltpu.VMEM((RPW_ * 16,), jnp.float32),
            pltpu.VMEM((RPW_ * 16,), jnp.float32),
            pltpu.SemaphoreType.DMA,
            pltpu.SemaphoreType.DMA,
        ],
    )


def _fold128(v):
    acc = v[:, 0:128]
    for k in range(1, v.shape[1] // 128):
        acc = acc + v[:, k * 128:(k + 1) * 128]
    return acc


def _tc_body(label_ref, cos_ref, out_ref, sacc_ref, tacc_ref, *, nblk):
    j = pl.program_id(0)

    @pl.when(j == 0)
    def _init():
        sacc_ref[...] = jnp.zeros_like(sacc_ref)
        tacc_ref[...] = jnp.zeros_like(tacc_ref)

    x = cos_ref[...]
    lbl_loc = label_ref[...] - j * BC_
    is_t = jax.lax.broadcasted_iota(jnp.int32, (RTC_, BC_), 1) == lbl_loc
    e = jnp.exp2(x * K_ - K_)

    @pl.when(j < nblk - 1)
    def _main():
        sacc_ref[...] += _fold128(e)
        tacc_ref[...] += _fold128(jnp.where(is_t, x, 0.0))

    @pl.when(j == nblk - 1)
    def _last():
        valid = jax.lax.broadcasted_iota(jnp.int32, (RTC_, BC_), 1) < (C_ - (nblk - 1) * BC_)
        sacc_ref[...] += _fold128(jnp.where(valid, e, 0.0))
        tacc_ref[...] += _fold128(jnp.where(is_t, x, 0.0))
        s = jnp.sum(sacc_ref[...], axis=1, keepdims=True)
        t = jnp.sum(tacc_ref[...], axis=1, keepdims=True)
        tm = (t - MARGIN_) * SCALE_
        s = s - jnp.exp2(t * K_ - K_) + jnp.exp2(tm * LOG2E_ - K_)
        nll = SCALE_ + jnp.log(s) - tm
        out_ref[0, 0] = jnp.sum(nll)


def _tc_main(cosine, label):
    nblk = pl.cdiv(C_, BC_)
    return pl.pallas_call(
        functools.partial(_tc_body, nblk=nblk),
        grid=(nblk,),
        in_specs=[
            pl.BlockSpec((RTC_, 1), lambda j: (0, 0)),
            pl.BlockSpec((RTC_, BC_), lambda j: (0, j)),
        ],
        out_specs=pl.BlockSpec((1, 1), lambda j: (0, 0), memory_space=pltpu.SMEM),
        out_shape=jax.ShapeDtypeStruct((1, 1), jnp.float32),
        scratch_shapes=[
            pltpu.VMEM((RTC_, 128), jnp.float32),
            pltpu.VMEM((RTC_, 128), jnp.float32),
        ],
    )(label[:RTC_].reshape(RTC_, 1), cosine)


def _combine_body(s_slab_ref, t_slab_ref, lbl_ref, cos_ref, tc_part_ref, loss_ref):
    x = cos_ref[...]
    col = NTBLK_ * BC_ + jax.lax.broadcasted_iota(jnp.int32, (RSC_, BC_), 1)
    lbl = lbl_ref[...]
    e = jnp.where(col < C_, jnp.exp2(x * K_ - K_), 0.0)
    s_tail = jnp.sum(e, axis=1, keepdims=True)
    t_tail = jnp.sum(jnp.where(col == lbl, x, 0.0), axis=1, keepdims=True)
    s = jnp.sum(s_slab_ref[...], axis=1, keepdims=True) + s_tail
    t = t_slab_ref[:, 0:1] + t_tail
    tm = (t - MARGIN_) * SCALE_
    s = s - jnp.exp2(t * K_ - K_) + jnp.exp2(tm * LOG2E_ - K_)
    nll = SCALE_ + jnp.log(s) - tm
    loss_ref[0, 0] = (jnp.sum(nll) + tc_part_ref[0, 0]) / B_


def _combine(s_slab, t_slab, label, cosine, tc_part):
    return pl.pallas_call(
        _combine_body,
        grid=(1,),
        in_specs=[
            pl.BlockSpec((RSC_, 16), lambda j: (0, 0)),
            pl.BlockSpec((RSC_, 16), lambda j: (0, 0)),
            pl.BlockSpec((RSC_, 1), lambda j: (RTC_ // RSC_, 0)),
            pl.BlockSpec((RSC_, BC_), lambda j: (RTC_ // RSC_, NTBLK_)),
            pl.BlockSpec((1, 1), lambda j: (0, 0), memory_space=pltpu.SMEM),
        ],
        out_specs=pl.BlockSpec((1, 1), lambda j: (0, 0), memory_space=pltpu.SMEM),
        out_shape=jax.ShapeDtypeStruct((1, 1), jnp.float32),
    )(s_slab, t_slab, label, cosine, tc_part)


def kernel(cosine, label):
    sc_fn = _make_sc()
    s_slab, t_slab = sc_fn(cosine, label[RTC_:])
    tc_part = _tc_main(cosine, label)
    loss = _combine(
        s_slab.reshape(RSC_, 16), t_slab.reshape(RSC_, 16),
        label.reshape(B_, 1), cosine, tc_part)
    return loss[0, 0]


# speedup vs baseline: 9.5512x; 3.2193x over previous
import functools

import jax
import jax.numpy as jnp
from jax.experimental import pallas as pl
from jax.experimental.pallas import tpu as pltpu

SCALE_ = 30.0
MARGIN_ = 0.2
B_ = 1024
C_ = 100000
BCT_ = 2048
LOG2E_ = 1.4426950408889634
K_ = SCALE_ * LOG2E_


def _fold8(v):
    acc = v[0:8, :]
    for k in range(1, v.shape[0] // 8):
        acc = acc + v[k * 8:(k + 1) * 8, :]
    return acc


def _loss_body(label_ref, cos_ref, loss_ref, sacc_ref, tacc_ref, *, nblk):
    j = pl.program_id(0)

    @pl.when(j == 0)
    def _init():
        sacc_ref[...] = jnp.zeros_like(sacc_ref)
        tacc_ref[...] = jnp.zeros_like(tacc_ref)

    x = cos_ref[...]
    cls = j * BCT_ + jax.lax.broadcasted_iota(jnp.int32, (BCT_, B_), 0)
    is_t = cls == label_ref[...]
    e = jnp.exp2(x * K_ - K_)

    @pl.when(j < nblk - 1)
    def _main():
        sacc_ref[...] += _fold8(e)
        tacc_ref[...] += _fold8(jnp.where(is_t, x, 0.0))

    @pl.when(j == nblk - 1)
    def _last():
        valid = cls < C_
        sacc_ref[...] += _fold8(jnp.where(valid, e, 0.0))
        tacc_ref[...] += _fold8(jnp.where(is_t, x, 0.0))
        s = jnp.sum(sacc_ref[...], axis=0, keepdims=True)
        t = jnp.sum(tacc_ref[...], axis=0, keepdims=True)
        tm = (t - MARGIN_) * SCALE_
        s = s - jnp.exp2(t * K_ - K_) + jnp.exp2(tm * LOG2E_ - K_)
        nll = SCALE_ + jnp.log(s) - tm
        loss_ref[0, 0] = jnp.sum(nll) / B_


def kernel(cosine, label):
    cos_t = cosine.T
    nblk = pl.cdiv(C_, BCT_)
    loss = pl.pallas_call(
        functools.partial(_loss_body, nblk=nblk),
        grid=(nblk,),
        in_specs=[
            pl.BlockSpec((1, B_), lambda j: (0, 0)),
            pl.BlockSpec((BCT_, B_), lambda j: (j, 0)),
        ],
        out_specs=pl.BlockSpec((1, 1), lambda j: (0, 0), memory_space=pltpu.SMEM),
        out_shape=jax.ShapeDtypeStruct((1, 1), jnp.float32),
        scratch_shapes=[
            pltpu.VMEM((8, B_), jnp.float32),
            pltpu.VMEM((8, B_), jnp.float32),
        ],
    )(label.reshape(1, B_), cos_t)
    return loss[0, 0]
